# baseline (speedup 1.0000x reference)
import functools

import jax
import jax.numpy as jnp
from jax import lax
from jax.experimental import pallas as pl
from jax.experimental.pallas import tpu as pltpu
from jax.experimental.pallas import tpu_sc as plsc

_N_CODES = 1024
_D = 256
_BETA = 0.25
_DECAY = 0.97
_N_ROWS = 16384
_BLK = 4096
_GRID = _N_ROWS // _BLK
_NC, _NS = 2, 16
_NW = _NC * _NS
_RPW = _N_ROWS // _NW
_CHUNK = 128
_CBLK = 2048
_CGRID = _N_ROWS // _CBLK
_NCHUNK = _RPW // _CHUNK


def _assign_body(z_ref, cb_ref, ema_w_ref, codes_ref, cbpack_ref,
                 cbn_ref, dw_ref):
    i = pl.program_id(0)

    @pl.when(i == 0)
    def _():
        cb = cb_ref[...]
        nrm = jnp.sqrt(jnp.sum(cb * cb, axis=1, keepdims=True))
        cbn_ref[...] = (cb / jnp.maximum(nrm, 1e-12)) * (-2.0)
        dw_ref[...] = jnp.zeros_like(dw_ref)

    z = z_ref[...]
    zn = z / jnp.maximum(jnp.sqrt(jnp.sum(z * z, axis=1, keepdims=True)), 1e-12)
    dots2 = lax.dot_general(zn, cbn_ref[...], (((1,), (1,)), ((), ())),
                            preferred_element_type=jnp.float32)
    d = 2.0 + dots2
    dmin = jnp.min(d, axis=1, keepdims=True)
    mask = d == dmin
    idxf = lax.broadcasted_iota(jnp.int32, (1, _N_CODES), 1).astype(jnp.float32)
    codes = jnp.min(jnp.where(mask, idxf, float(_N_CODES)),
                    axis=1).astype(jnp.int32)
    codes_ref[0, 0, :] = codes
    dwp = lax.dot_general(mask.astype(jnp.bfloat16), zn.astype(jnp.bfloat16),
                          (((0,), (0,)), ((), ())),
                          preferred_element_type=jnp.float32)
    dw_ref[...] += dwp

    @pl.when(i == _GRID - 1)
    def _():
        w = ema_w_ref[...] * _DECAY + (1.0 - _DECAY) * dw_ref[...]
        nrm = jnp.sqrt(jnp.sum(w * w, axis=1, keepdims=True))
        cbnew = w / jnp.maximum(nrm, 1e-12)
        lo = lax.bitcast_convert_type(
            cbnew[:, :128].astype(jnp.bfloat16), jnp.uint16).astype(jnp.uint32)
        hi = lax.bitcast_convert_type(
            cbnew[:, 128:].astype(jnp.bfloat16), jnp.uint16).astype(jnp.uint32)
        cbpack_ref[...] = (lo | (hi << 16)).astype(jnp.int32)


def _assign(z_e, codebook, ema_w):
    return pl.pallas_call(
        _assign_body,
        grid=(_GRID,),
        in_specs=[
            pl.BlockSpec((_BLK, _D), lambda i: (i, 0)),
            pl.BlockSpec((_N_CODES, _D), lambda i: (0, 0)),
            pl.BlockSpec((_N_CODES, _D), lambda i: (0, 0)),
        ],
        out_specs=[
            pl.BlockSpec((1, 1, _BLK), lambda i: (i, 0, 0)),
            pl.BlockSpec((_N_CODES, _D // 2), lambda i: (0, 0)),
        ],
        out_shape=[
            jax.ShapeDtypeStruct((_GRID, 1, _BLK), jnp.int32),
            jax.ShapeDtypeStruct((_N_CODES, _D // 2), jnp.int32),
        ],
        scratch_shapes=[
            pltpu.VMEM((_N_CODES, _D), jnp.float32),
            pltpu.VMEM((_N_CODES, _D), jnp.float32),
        ],
    )(z_e, codebook, ema_w)


def _gather_body(codes_hbm, cb_hbm, zq_hbm, idx_v, rows0, rows1, gs0, gs1, ws0, ws1):
    c = lax.axis_index("c")
    s = lax.axis_index("s")
    wid = s * _NC + c
    base = wid * _RPW
    rows = (rows0, rows1)
    gsem = (gs0, gs1)
    wsem = (ws0, ws1)
    pltpu.sync_copy(codes_hbm.at[wid], idx_v)
    gathers = [None] * _NCHUNK
    writes = [None] * _NCHUNK
    for k in range(2):
        gathers[k] = pltpu.async_copy(
            cb_hbm.at[idx_v.at[k]], rows[k], gsem[k])
    for k in range(_NCHUNK):
        b = k % 2
        gathers[k].wait()
        writes[k] = pltpu.async_copy(
            rows[b], zq_hbm.at[pl.ds(base + k * _CHUNK, _CHUNK)], wsem[b])
        if k + 2 < _NCHUNK:
            writes[k].wait()
            gathers[k + 2] = pltpu.async_copy(
                cb_hbm.at[idx_v.at[k + 2]], rows[b], gsem[b])
    writes[_NCHUNK - 2].wait()
    writes[_NCHUNK - 1].wait()


def _gather(codes, cbpack):
    mesh = plsc.VectorSubcoreMesh(core_axis_name="c", subcore_axis_name="s")
    run = functools.partial(
        pl.kernel,
        out_type=jax.ShapeDtypeStruct((_N_ROWS, _D // 2), jnp.int32),
        mesh=mesh,
        scratch_types=[
            pltpu.VMEM((_NCHUNK, _CHUNK), jnp.int32),
            pltpu.VMEM((_CHUNK, _D // 2), jnp.int32),
            pltpu.VMEM((_CHUNK, _D // 2), jnp.int32),
            pltpu.SemaphoreType.DMA,
            pltpu.SemaphoreType.DMA,
            pltpu.SemaphoreType.DMA,
            pltpu.SemaphoreType.DMA,
        ],
    )(_gather_body)
    return run(codes.reshape(_NW, _NCHUNK, _CHUNK), cbpack)


def _out_body(ze_ref, zq_ref, out_ref, loss_ref, acc_ref):
    i = pl.program_id(0)
    zp = zq_ref[...]
    zq_lo = lax.bitcast_convert_type(zp << 16, jnp.float32)
    zq_hi = lax.bitcast_convert_type(zp & jnp.int32(-65536), jnp.float32)
    ze_lo = ze_ref[:, :128]
    ze_hi = ze_ref[:, 128:]
    t_lo = zq_lo - ze_lo
    t_hi = zq_hi - ze_hi
    out_ref[:, :128] = ze_lo + t_lo
    out_ref[:, 128:] = ze_hi + t_hi

    @pl.when(i == 0)
    def _():
        acc_ref[...] = jnp.zeros_like(acc_ref)

    acc_ref[...] += jnp.concatenate(
        [jnp.sum(t_lo * t_lo, axis=0, keepdims=True),
         jnp.sum(t_hi * t_hi, axis=0, keepdims=True)], axis=1)

    @pl.when(i == _CGRID - 1)
    def _():
        loss_ref[0, 0] = _BETA * jnp.sum(acc_ref[...]) / (_N_ROWS * _D)


def _finalize(z_e, zq):
    return pl.pallas_call(
        _out_body,
        grid=(_CGRID,),
        in_specs=[
            pl.BlockSpec((_CBLK, _D), lambda i: (i, 0)),
            pl.BlockSpec((_CBLK, _D // 2), lambda i: (i, 0)),
        ],
        out_specs=[
            pl.BlockSpec((_CBLK, _D), lambda i: (i, 0)),
            pl.BlockSpec((1, 1), lambda i: (0, 0), memory_space=pltpu.SMEM),
        ],
        out_shape=[
            jax.ShapeDtypeStruct((_N_ROWS, _D), jnp.float32),
            jax.ShapeDtypeStruct((1, 1), jnp.float32),
        ],
        scratch_shapes=[pltpu.VMEM((1, _D), jnp.float32)],
    )(z_e, zq)


def kernel(z_e, codebook, ema_cluster_size, ema_w):
    del ema_cluster_size
    codes3, cbpack = _assign(z_e, codebook, ema_w)
    codes = codes3.reshape(_N_ROWS)
    zqp = _gather(codes, cbpack)
    zq_out, loss = _finalize(z_e, zqp)
    return (zq_out, codes, loss.reshape(()))


# speedup vs baseline: 1.0763x; 1.0763x over previous
import functools

import jax
import jax.numpy as jnp
from jax import lax
from jax.experimental import pallas as pl
from jax.experimental.pallas import tpu as pltpu
from jax.experimental.pallas import tpu_sc as plsc

_N_CODES = 1024
_D = 256
_BETA = 0.25
_DECAY = 0.97
_N_ROWS = 16384
_BLK = 4096
_GRID = _N_ROWS // _BLK
_NC, _NS = 2, 16
_NW = _NC * _NS
_HALF = _N_ROWS // 2
_SC_RPW = _HALF // _NW
_CHUNK = 128
_NCHUNK = _SC_RPW // _CHUNK
_CBLK = 2048
_CGRID_H = _HALF // _CBLK


def _assign_body(z_ref, cb_ref, ema_w_ref, codes_ref, cbpack_ref,
                 cbn_ref, dw_ref):
    i = pl.program_id(0)

    @pl.when(i == 0)
    def _():
        cb = cb_ref[...]
        nrm = jnp.sqrt(jnp.sum(cb * cb, axis=1, keepdims=True))
        cbn_ref[...] = (cb / jnp.maximum(nrm, 1e-12)) * (-2.0)
        dw_ref[...] = jnp.zeros_like(dw_ref)

    z = z_ref[...]
    zn = z / jnp.maximum(jnp.sqrt(jnp.sum(z * z, axis=1, keepdims=True)), 1e-12)
    dots2 = lax.dot_general(zn, cbn_ref[...], (((1,), (1,)), ((), ())),
                            preferred_element_type=jnp.float32)
    d = 2.0 + dots2
    dmin = jnp.min(d, axis=1, keepdims=True)
    mask = d == dmin
    idxf = lax.broadcasted_iota(jnp.int32, (1, _N_CODES), 1).astype(jnp.float32)
    codes = jnp.min(jnp.where(mask, idxf, float(_N_CODES)),
                    axis=1).astype(jnp.int32)
    codes_ref[0, 0, :] = codes
    dwp = lax.dot_general(mask.astype(jnp.bfloat16), zn.astype(jnp.bfloat16),
                          (((0,), (0,)), ((), ())),
                          preferred_element_type=jnp.float32)
    dw_ref[...] += dwp

    @pl.when(i == _GRID - 1)
    def _():
        w = ema_w_ref[...] * _DECAY + (1.0 - _DECAY) * dw_ref[...]
        nrm = jnp.sqrt(jnp.sum(w * w, axis=1, keepdims=True))
        cbnew = w / jnp.maximum(nrm, 1e-12)
        lo = lax.bitcast_convert_type(
            cbnew[:, :128].astype(jnp.bfloat16), jnp.uint16).astype(jnp.uint32)
        hi = lax.bitcast_convert_type(
            cbnew[:, 128:].astype(jnp.bfloat16), jnp.uint16).astype(jnp.uint32)
        cbpack_ref[...] = (lo | (hi << 16)).astype(jnp.int32)


def _assign(z_e, codebook, ema_w):
    return pl.pallas_call(
        _assign_body,
        grid=(_GRID,),
        in_specs=[
            pl.BlockSpec((_BLK, _D), lambda i: (i, 0)),
            pl.BlockSpec((_N_CODES, _D), lambda i: (0, 0)),
            pl.BlockSpec((_N_CODES, _D), lambda i: (0, 0)),
        ],
        out_specs=[
            pl.BlockSpec((1, 1, _BLK), lambda i: (i, 0, 0)),
            pl.BlockSpec((_N_CODES, _D // 2), lambda i: (0, 0)),
        ],
        out_shape=[
            jax.ShapeDtypeStruct((_GRID, 1, _BLK), jnp.int32),
            jax.ShapeDtypeStruct((_N_CODES, _D // 2), jnp.int32),
        ],
        scratch_shapes=[
            pltpu.VMEM((_N_CODES, _D), jnp.float32),
            pltpu.VMEM((_N_CODES, _D), jnp.float32),
        ],
    )(z_e, codebook, ema_w)


def _gather_body(codes_hbm, cb_hbm, zq_hbm, idx_v, rows0, rows1, gs0, gs1, ws0, ws1):
    c = lax.axis_index("c")
    s = lax.axis_index("s")
    wid = s * _NC + c
    base = wid * _SC_RPW
    rows = (rows0, rows1)
    gsem = (gs0, gs1)
    wsem = (ws0, ws1)
    pltpu.sync_copy(codes_hbm.at[wid], idx_v)
    gathers = [None] * _NCHUNK
    writes = [None] * _NCHUNK
    for k in range(2):
        gathers[k] = pltpu.async_copy(
            cb_hbm.at[idx_v.at[k]], rows[k], gsem[k])
    for k in range(_NCHUNK):
        b = k % 2
        gathers[k].wait()
        writes[k] = pltpu.async_copy(
            rows[b], zq_hbm.at[pl.ds(base + k * _CHUNK, _CHUNK)], wsem[b])
        if k + 2 < _NCHUNK:
            writes[k].wait()
            gathers[k + 2] = pltpu.async_copy(
                cb_hbm.at[idx_v.at[k + 2]], rows[b], gsem[b])
    writes[_NCHUNK - 2].wait()
    writes[_NCHUNK - 1].wait()


def _gather(codes_sc, cbpack):
    mesh = plsc.VectorSubcoreMesh(core_axis_name="c", subcore_axis_name="s")
    run = functools.partial(
        pl.kernel,
        out_type=jax.ShapeDtypeStruct((_HALF, _D // 2), jnp.int32),
        mesh=mesh,
        scratch_types=[
            pltpu.VMEM((_NCHUNK, _CHUNK), jnp.int32),
            pltpu.VMEM((_CHUNK, _D // 2), jnp.int32),
            pltpu.VMEM((_CHUNK, _D // 2), jnp.int32),
            pltpu.SemaphoreType.DMA,
            pltpu.SemaphoreType.DMA,
            pltpu.SemaphoreType.DMA,
            pltpu.SemaphoreType.DMA,
        ],
    )(_gather_body)
    return run(codes_sc.reshape(_NW, _NCHUNK, _CHUNK), cbpack)


def _half2_body(ze_ref, codes_ref, cbp_ref, out_ref, accb_ref, cbbf_ref, acc_ref):
    i = pl.program_id(0)

    @pl.when(i == 0)
    def _():
        cbp = cbp_ref[...]
        lo = lax.bitcast_convert_type(cbp << 16, jnp.float32)
        hi = lax.bitcast_convert_type(cbp & jnp.int32(-65536), jnp.float32)
        cbbf_ref[...] = jnp.concatenate(
            [lo.astype(jnp.bfloat16), hi.astype(jnp.bfloat16)], axis=1)
        acc_ref[...] = jnp.zeros_like(acc_ref)

    codes_blk = codes_ref[0, 0, :]
    oh = (codes_blk[:, None] ==
          lax.broadcasted_iota(jnp.int32, (1, _N_CODES), 1)).astype(jnp.bfloat16)
    zq = lax.dot_general(oh, cbbf_ref[...], (((1,), (0,)), ((), ())),
                         preferred_element_type=jnp.float32)
    ze = ze_ref[...]
    t = zq - ze
    out_ref[...] = ze + t
    acc_ref[...] += jnp.sum(t * t, axis=0, keepdims=True)

    @pl.when(i == _CGRID_H - 1)
    def _():
        accb_ref[...] = acc_ref[...]


def _half2(z_e, codes_b, cbpack):
    return pl.pallas_call(
        _half2_body,
        grid=(_CGRID_H,),
        in_specs=[
            pl.BlockSpec((_CBLK, _D), lambda i: (i + _CGRID_H, 0)),
            pl.BlockSpec((1, 1, _CBLK), lambda i: (i, 0, 0)),
            pl.BlockSpec((_N_CODES, _D // 2), lambda i: (0, 0)),
        ],
        out_specs=[
            pl.BlockSpec((_CBLK, _D), lambda i: (i + _CGRID_H, 0)),
            pl.BlockSpec((1, _D), lambda i: (0, 0)),
        ],
        out_shape=[
            jax.ShapeDtypeStruct((_N_ROWS, _D), jnp.float32),
            jax.ShapeDtypeStruct((1, _D), jnp.float32),
        ],
        scratch_shapes=[
            pltpu.VMEM((_N_CODES, _D), jnp.bfloat16),
            pltpu.VMEM((1, _D), jnp.float32),
        ],
    )(z_e, codes_b, cbpack)


def _half1_body(ze_ref, zq_ref, accb_ref, prev_ref, out_ref, loss_ref, acc_ref):
    i = pl.program_id(0)
    zp = zq_ref[...]
    zq_lo = lax.bitcast_convert_type(zp << 16, jnp.float32)
    zq_hi = lax.bitcast_convert_type(zp & jnp.int32(-65536), jnp.float32)
    ze_lo = ze_ref[:, :128]
    ze_hi = ze_ref[:, 128:]
    t_lo = zq_lo - ze_lo
    t_hi = zq_hi - ze_hi
    out_ref[:, :128] = ze_lo + t_lo
    out_ref[:, 128:] = ze_hi + t_hi

    @pl.when(i == 0)
    def _():
        acc_ref[...] = jnp.zeros_like(acc_ref)

    acc_ref[...] += jnp.concatenate(
        [jnp.sum(t_lo * t_lo, axis=0, keepdims=True),
         jnp.sum(t_hi * t_hi, axis=0, keepdims=True)], axis=1)

    @pl.when(i == _CGRID_H - 1)
    def _():
        loss_ref[0, 0] = _BETA * (jnp.sum(acc_ref[...]) +
                                  jnp.sum(accb_ref[...])) / (_N_ROWS * _D)


def _half1(z_e, zqp, accb, out1):
    return pl.pallas_call(
        _half1_body,
        grid=(_CGRID_H,),
        in_specs=[
            pl.BlockSpec((_CBLK, _D), lambda i: (i, 0)),
            pl.BlockSpec((_CBLK, _D // 2), lambda i: (i, 0)),
            pl.BlockSpec((1, _D), lambda i: (0, 0)),
            pl.BlockSpec((8, 128), lambda i: (0, 0)),
        ],
        out_specs=[
            pl.BlockSpec((_CBLK, _D), lambda i: (i, 0)),
            pl.BlockSpec((1, 1), lambda i: (0, 0), memory_space=pltpu.SMEM),
        ],
        out_shape=[
            jax.ShapeDtypeStruct((_N_ROWS, _D), jnp.float32),
            jax.ShapeDtypeStruct((1, 1), jnp.float32),
        ],
        scratch_shapes=[pltpu.VMEM((1, _D), jnp.float32)],
        input_output_aliases={3: 0},
    )(z_e, zqp, accb, out1)


def kernel(z_e, codebook, ema_cluster_size, ema_w):
    del ema_cluster_size
    codes3, cbpack = _assign(z_e, codebook, ema_w)
    codes = codes3.reshape(_N_ROWS)
    zqp = _gather(codes[:_HALF], cbpack)
    codes_b = codes[_HALF:].reshape(_CGRID_H, 1, _CBLK)
    out1, accb = _half2(z_e, codes_b, cbpack)
    zq_out, loss = _half1(z_e, zqp, accb, out1)
    return (zq_out, codes, loss.reshape(()))


# speedup vs baseline: 1.0826x; 1.0058x over previous
import functools

import jax
import jax.numpy as jnp
from jax import lax
from jax.experimental import pallas as pl
from jax.experimental.pallas import tpu as pltpu
from jax.experimental.pallas import tpu_sc as plsc

_N_CODES = 1024
_D = 256
_BETA = 0.25
_DECAY = 0.97
_N_ROWS = 16384
_BLK = 4096
_GRID = _N_ROWS // _BLK
_NC, _NS = 2, 16
_NW = _NC * _NS
_HALF = _N_ROWS // 2
_SC_RPW = _HALF // _NW
_CHUNK = 128
_NCHUNK = _SC_RPW // _CHUNK
_CBLK = 2048
_CGRID_H = _HALF // _CBLK


def _assign_body(z_ref, cb_ref, ema_w_ref, codes_ref, cbpack_ref,
                 cbn_ref, dw_ref):
    i = pl.program_id(0)

    @pl.when(i == 0)
    def _():
        cb = cb_ref[...]
        nrm = jnp.sqrt(jnp.sum(cb * cb, axis=1, keepdims=True))
        cbn_ref[...] = (cb / jnp.maximum(nrm, 1e-12)) * (-2.0)
        dw_ref[...] = jnp.zeros_like(dw_ref)

    z = z_ref[...]
    zn = z / jnp.maximum(jnp.sqrt(jnp.sum(z * z, axis=1, keepdims=True)), 1e-12)
    dots2 = lax.dot_general(zn, cbn_ref[...], (((1,), (1,)), ((), ())),
                            preferred_element_type=jnp.float32)
    d = 2.0 + dots2
    dmin = jnp.min(d, axis=1, keepdims=True)
    mask = d == dmin
    idxf = lax.broadcasted_iota(jnp.int32, (1, _N_CODES), 1).astype(jnp.float32)
    codes = jnp.min(jnp.where(mask, idxf, float(_N_CODES)),
                    axis=1).astype(jnp.int32)
    codes_ref[0, 0, :] = codes
    dwp = lax.dot_general(mask.astype(jnp.bfloat16), zn.astype(jnp.bfloat16),
                          (((0,), (0,)), ((), ())),
                          preferred_element_type=jnp.float32)
    dw_ref[...] += dwp

    @pl.when(i == _GRID - 1)
    def _():
        w = ema_w_ref[...] * _DECAY + (1.0 - _DECAY) * dw_ref[...]
        nrm = jnp.sqrt(jnp.sum(w * w, axis=1, keepdims=True))
        cbnew = w / jnp.maximum(nrm, 1e-12)
        lo = lax.bitcast_convert_type(
            cbnew[:, :128].astype(jnp.bfloat16), jnp.uint16).astype(jnp.uint32)
        hi = lax.bitcast_convert_type(
            cbnew[:, 128:].astype(jnp.bfloat16), jnp.uint16).astype(jnp.uint32)
        cbpack_ref[...] = (lo | (hi << 16)).astype(jnp.int32)


def _assign(z_e, codebook, ema_w):
    return pl.pallas_call(
        _assign_body,
        grid=(_GRID,),
        in_specs=[
            pl.BlockSpec((_BLK, _D), lambda i: (i, 0)),
            pl.BlockSpec((_N_CODES, _D), lambda i: (0, 0)),
            pl.BlockSpec((_N_CODES, _D), lambda i: (0, 0)),
        ],
        out_specs=[
            pl.BlockSpec((1, 1, _BLK), lambda i: (i, 0, 0)),
            pl.BlockSpec((_N_CODES, _D // 2), lambda i: (0, 0)),
        ],
        out_shape=[
            jax.ShapeDtypeStruct((_GRID, 1, _BLK), jnp.int32),
            jax.ShapeDtypeStruct((_N_CODES, _D // 2), jnp.int32),
        ],
        scratch_shapes=[
            pltpu.VMEM((_N_CODES, _D), jnp.float32),
            pltpu.VMEM((_N_CODES, _D), jnp.float32),
        ],
    )(z_e, codebook, ema_w)


def _gather_body(codes_hbm, cb_hbm, zq_hbm, idx_v, rows0, rows1, gs0, gs1, ws0, ws1):
    c = lax.axis_index("c")
    s = lax.axis_index("s")
    base = c * (_NS * _SC_RPW) + s * _SC_RPW
    rows = (rows0, rows1)
    gsem = (gs0, gs1)
    wsem = (ws0, ws1)
    pltpu.sync_copy(codes_hbm.at[c, 0, pl.ds(s * _SC_RPW, _SC_RPW)], idx_v)
    gathers = [None] * _NCHUNK
    writes = [None] * _NCHUNK
    for k in range(2):
        gathers[k] = pltpu.async_copy(
            cb_hbm.at[idx_v.at[pl.ds(k * _CHUNK, _CHUNK)]], rows[k], gsem[k])
    for k in range(_NCHUNK):
        b = k % 2
        gathers[k].wait()
        writes[k] = pltpu.async_copy(
            rows[b], zq_hbm.at[pl.ds(base + k * _CHUNK, _CHUNK)], wsem[b])
        if k + 2 < _NCHUNK:
            writes[k].wait()
            gathers[k + 2] = pltpu.async_copy(
                cb_hbm.at[idx_v.at[pl.ds((k + 2) * _CHUNK, _CHUNK)]], rows[b], gsem[b])
    writes[_NCHUNK - 2].wait()
    writes[_NCHUNK - 1].wait()


def _gather(codes3, cbpack):
    mesh = plsc.VectorSubcoreMesh(core_axis_name="c", subcore_axis_name="s")
    run = functools.partial(
        pl.kernel,
        out_type=jax.ShapeDtypeStruct((_HALF, _D // 2), jnp.int32),
        mesh=mesh,
        scratch_types=[
            pltpu.VMEM((_SC_RPW,), jnp.int32),
            pltpu.VMEM((_CHUNK, _D // 2), jnp.int32),
            pltpu.VMEM((_CHUNK, _D // 2), jnp.int32),
            pltpu.SemaphoreType.DMA,
            pltpu.SemaphoreType.DMA,
            pltpu.SemaphoreType.DMA,
            pltpu.SemaphoreType.DMA,
        ],
    )(_gather_body)
    return run(codes3, cbpack)


def _half2_body(ze_ref, codes_ref, cbp_ref, out_ref, accb_ref, cbbf_ref, acc_ref):
    i = pl.program_id(0)

    @pl.when(i == 0)
    def _():
        cbp = cbp_ref[...]
        lo = lax.bitcast_convert_type(cbp << 16, jnp.float32)
        hi = lax.bitcast_convert_type(cbp & jnp.int32(-65536), jnp.float32)
        cbbf_ref[...] = jnp.concatenate(
            [lo.astype(jnp.bfloat16), hi.astype(jnp.bfloat16)], axis=1)
        acc_ref[...] = jnp.zeros_like(acc_ref)

    codes_blk = codes_ref[0, 0, :]
    oh = (codes_blk[:, None] ==
          lax.broadcasted_iota(jnp.int32, (1, _N_CODES), 1)).astype(jnp.bfloat16)
    zq = lax.dot_general(oh, cbbf_ref[...], (((1,), (0,)), ((), ())),
                         preferred_element_type=jnp.float32)
    ze = ze_ref[...]
    t = zq - ze
    out_ref[...] = ze + t
    acc_ref[...] += jnp.sum(t * t, axis=0, keepdims=True)

    @pl.when(i == _CGRID_H - 1)
    def _():
        accb_ref[...] = acc_ref[...]


def _half2(z_e, codes_b, cbpack):
    return pl.pallas_call(
        _half2_body,
        grid=(_CGRID_H,),
        in_specs=[
            pl.BlockSpec((_CBLK, _D), lambda i: (i + _CGRID_H, 0)),
            pl.BlockSpec((1, 1, _CBLK), lambda i: (2 + i // 2, 0, i % 2)),
            pl.BlockSpec((_N_CODES, _D // 2), lambda i: (0, 0)),
        ],
        out_specs=[
            pl.BlockSpec((_CBLK, _D), lambda i: (i + _CGRID_H, 0)),
            pl.BlockSpec((1, _D), lambda i: (0, 0)),
        ],
        out_shape=[
            jax.ShapeDtypeStruct((_N_ROWS, _D), jnp.float32),
            jax.ShapeDtypeStruct((1, _D), jnp.float32),
        ],
        scratch_shapes=[
            pltpu.VMEM((_N_CODES, _D), jnp.bfloat16),
            pltpu.VMEM((1, _D), jnp.float32),
        ],
    )(z_e, codes_b, cbpack)


def _half1_body(ze_ref, zq_ref, accb_ref, prev_ref, out_ref, loss_ref, acc_ref):
    i = pl.program_id(0)
    zp = zq_ref[...]
    zq_lo = lax.bitcast_convert_type(zp << 16, jnp.float32)
    zq_hi = lax.bitcast_convert_type(zp & jnp.int32(-65536), jnp.float32)
    ze_lo = ze_ref[:, :128]
    ze_hi = ze_ref[:, 128:]
    t_lo = zq_lo - ze_lo
    t_hi = zq_hi - ze_hi
    out_ref[:, :128] = ze_lo + t_lo
    out_ref[:, 128:] = ze_hi + t_hi

    @pl.when(i == 0)
    def _():
        acc_ref[...] = jnp.zeros_like(acc_ref)

    acc_ref[...] += jnp.concatenate(
        [jnp.sum(t_lo * t_lo, axis=0, keepdims=True),
         jnp.sum(t_hi * t_hi, axis=0, keepdims=True)], axis=1)

    @pl.when(i == _CGRID_H - 1)
    def _():
        loss_ref[0, 0] = _BETA * (jnp.sum(acc_ref[...]) +
                                  jnp.sum(accb_ref[...])) / (_N_ROWS * _D)


def _half1(z_e, zqp, accb, out1):
    return pl.pallas_call(
        _half1_body,
        grid=(_CGRID_H,),
        in_specs=[
            pl.BlockSpec((_CBLK, _D), lambda i: (i, 0)),
            pl.BlockSpec((_CBLK, _D // 2), lambda i: (i, 0)),
            pl.BlockSpec((1, _D), lambda i: (0, 0)),
            pl.BlockSpec((8, 128), lambda i: (0, 0)),
        ],
        out_specs=[
            pl.BlockSpec((_CBLK, _D), lambda i: (i, 0)),
            pl.BlockSpec((1, 1), lambda i: (0, 0), memory_space=pltpu.SMEM),
        ],
        out_shape=[
            jax.ShapeDtypeStruct((_N_ROWS, _D), jnp.float32),
            jax.ShapeDtypeStruct((1, 1), jnp.float32),
        ],
        scratch_shapes=[pltpu.VMEM((1, _D), jnp.float32)],
        input_output_aliases={3: 0},
    )(z_e, zqp, accb, out1)


def kernel(z_e, codebook, ema_cluster_size, ema_w):
    del ema_cluster_size
    codes3, cbpack = _assign(z_e, codebook, ema_w)
    zqp = _gather(codes3, cbpack)
    out1, accb = _half2(z_e, codes3, cbpack)
    zq_out, loss = _half1(z_e, zqp, accb, out1)
    return (zq_out, codes3.reshape(_N_ROWS), loss.reshape(()))
